# baseline (speedup 1.0000x reference)
import functools

import jax
import jax.numpy as jnp
from jax.experimental import pallas as pl
from jax.experimental.pallas import tpu as pltpu

D_MODEL = 2048
N_EXP = 16
BT = 2048
KC = 4


def _router_block(x_ref, wt_ref, w_ref, i_ref, acc_ref, aux_ref, logits_ref,
                  *, nblocks, inv_t):
    step = pl.program_id(0)
    kstep = pl.program_id(1)

    partial = jnp.dot(x_ref[...], wt_ref[...], preferred_element_type=jnp.float32)

    @pl.when(kstep == 0)
    def _():
        logits_ref[...] = partial

    @pl.when(kstep != 0)
    def _():
        logits_ref[...] += partial

    @pl.when(kstep == KC - 1)
    def _():
        logits = logits_ref[...]
        ids = jax.lax.broadcasted_iota(jnp.int32, logits.shape, 1)
        m1 = jnp.max(logits, axis=1, keepdims=True)
        i1 = jnp.min(jnp.where(logits == m1, ids, N_EXP), axis=1, keepdims=True)
        masked = jnp.where(ids == i1, -jnp.inf, logits)
        m2 = jnp.max(masked, axis=1, keepdims=True)
        i2 = jnp.min(jnp.where(masked == m2, ids, N_EXP), axis=1, keepdims=True)

        t = jnp.exp(m2 - m1)
        w1 = 1.0 / (1.0 + t)
        w2 = t / (1.0 + t)
        w_ref[...] = jnp.concatenate([w1, w2], axis=1)
        i_ref[...] = jnp.concatenate([i1, i2], axis=1).astype(jnp.int32)

        p = jnp.exp(logits - m1)
        probs = p / jnp.sum(p, axis=1, keepdims=True)
        part = jnp.sum(probs, axis=0, keepdims=True)

        @pl.when(step == 0)
        def _():
            acc_ref[...] = jnp.zeros_like(acc_ref)

        acc_ref[...] += part

        @pl.when(step == nblocks - 1)
        def _():
            mean = acc_ref[...] * inv_t
            aux_ref[...] = jnp.sum(mean * mean, keepdims=True) * float(N_EXP)


def kernel(x, W):
    b, tt, d = x.shape
    total = b * tt
    xf = x.reshape(total, d)
    wt = W.T
    nblocks = total // BT
    kc = d // KC

    body = functools.partial(_router_block, nblocks=nblocks, inv_t=1.0 / total)
    weights, indices, _, aux = pl.pallas_call(
        body,
        grid=(nblocks, KC),
        in_specs=[
            pl.BlockSpec((BT, kc), lambda i, k: (i, k)),
            pl.BlockSpec((kc, N_EXP), lambda i, k: (k, 0)),
        ],
        out_specs=[
            pl.BlockSpec((BT, 2), lambda i, k: (i, 0)),
            pl.BlockSpec((BT, 2), lambda i, k: (i, 0)),
            pl.BlockSpec((1, N_EXP), lambda i, k: (0, 0)),
            pl.BlockSpec((1, 1), lambda i, k: (0, 0)),
        ],
        out_shape=[
            jax.ShapeDtypeStruct((total, 2), jnp.float32),
            jax.ShapeDtypeStruct((total, 2), jnp.int32),
            jax.ShapeDtypeStruct((1, N_EXP), jnp.float32),
            jax.ShapeDtypeStruct((1, 1), jnp.float32),
        ],
        scratch_shapes=[pltpu.VMEM((BT, N_EXP), jnp.float32)],
    )(xf, wt)

    return (weights.reshape(b, tt, 2), indices.reshape(b, tt, 2), aux[0, 0])


# speedup vs baseline: 1.3796x; 1.3796x over previous
import functools

import jax
import jax.numpy as jnp
from jax.experimental import pallas as pl

D_MODEL = 2048
N_EXP = 16
BT = 2048


def _router_block(x_ref, wt_ref, w_ref, i_ref, acc_ref, aux_ref, *, nblocks, inv_t):
    step = pl.program_id(0)

    logits = jnp.dot(x_ref[...], wt_ref[...], preferred_element_type=jnp.float32)

    ids = jax.lax.broadcasted_iota(jnp.int32, logits.shape, 1)
    m1 = jnp.max(logits, axis=1, keepdims=True)
    i1 = jnp.min(jnp.where(logits == m1, ids, N_EXP), axis=1, keepdims=True)
    masked = jnp.where(ids == i1, -jnp.inf, logits)
    m2 = jnp.max(masked, axis=1, keepdims=True)
    i2 = jnp.min(jnp.where(masked == m2, ids, N_EXP), axis=1, keepdims=True)

    t = jnp.exp(m2 - m1)
    w1 = 1.0 / (1.0 + t)
    w2 = t / (1.0 + t)
    w_ref[...] = jnp.concatenate([w1, w2], axis=1)
    i_ref[...] = jnp.concatenate([i1, i2], axis=1).astype(jnp.int32)

    p = jnp.exp(logits - m1)
    probs = p / jnp.sum(p, axis=1, keepdims=True)
    partial = jnp.sum(probs, axis=0, keepdims=True)

    @pl.when(step == 0)
    def _():
        acc_ref[...] = jnp.zeros_like(acc_ref)

    acc_ref[...] += partial

    @pl.when(step == nblocks - 1)
    def _():
        mean = acc_ref[...] * inv_t
        aux_ref[...] = jnp.sum(mean * mean, keepdims=True) * float(N_EXP)


def kernel(x, W):
    b, tt, d = x.shape
    total = b * tt
    xf = x.reshape(total, d)
    wt = W.T
    nblocks = total // BT

    body = functools.partial(_router_block, nblocks=nblocks, inv_t=1.0 / total)
    weights, indices, _, aux = pl.pallas_call(
        body,
        grid=(nblocks,),
        in_specs=[
            pl.BlockSpec((BT, d), lambda i: (i, 0)),
            pl.BlockSpec((d, N_EXP), lambda i: (0, 0)),
        ],
        out_specs=[
            pl.BlockSpec((BT, 2), lambda i: (i, 0)),
            pl.BlockSpec((BT, 2), lambda i: (i, 0)),
            pl.BlockSpec((1, N_EXP), lambda i: (0, 0)),
            pl.BlockSpec((1, 1), lambda i: (0, 0)),
        ],
        out_shape=[
            jax.ShapeDtypeStruct((total, 2), jnp.float32),
            jax.ShapeDtypeStruct((total, 2), jnp.int32),
            jax.ShapeDtypeStruct((1, N_EXP), jnp.float32),
            jax.ShapeDtypeStruct((1, 1), jnp.float32),
        ],
    )(xf, wt)

    return (weights.reshape(b, tt, 2), indices.reshape(b, tt, 2), aux[0, 0])


# speedup vs baseline: 1.4079x; 1.0205x over previous
import functools

import jax
import jax.numpy as jnp
from jax.experimental import pallas as pl

D_MODEL = 2048
N_EXP = 16
BT = 1024


def _top2_softmax(logits):
    ids = jax.lax.broadcasted_iota(jnp.int32, logits.shape, 1)
    m1 = jnp.max(logits, axis=1, keepdims=True)
    i1 = jnp.min(jnp.where(logits == m1, ids, N_EXP), axis=1, keepdims=True)
    masked = jnp.where(ids == i1, -jnp.inf, logits)
    m2 = jnp.max(masked, axis=1, keepdims=True)
    i2 = jnp.min(jnp.where(masked == m2, ids, N_EXP), axis=1, keepdims=True)

    t = jnp.exp(m2 - m1)
    w1 = 1.0 / (1.0 + t)
    w2 = t / (1.0 + t)
    w = jnp.concatenate([w1, w2], axis=1)
    idx = jnp.concatenate([i1, i2], axis=1).astype(jnp.int32)

    p = jnp.exp(logits - m1)
    probs = p / jnp.sum(p, axis=1, keepdims=True)
    part = jnp.sum(probs, axis=0, keepdims=True)
    return w, idx, part


def _router_block(xa_ref, xb_ref, wt_ref, w_ref, i_ref, acc_ref, aux_ref,
                  *, nblocks, inv_t):
    step = pl.program_id(0)
    wt = wt_ref[...]

    la = jnp.dot(xa_ref[0], wt, preferred_element_type=jnp.float32)
    wa, ia, pa = _top2_softmax(la)
    w_ref[0] = wa
    i_ref[0] = ia

    lb = jnp.dot(xb_ref[0], wt, preferred_element_type=jnp.float32)
    wb, ib, pb = _top2_softmax(lb)
    w_ref[1] = wb
    i_ref[1] = ib

    @pl.when(step == 0)
    def _():
        acc_ref[...] = jnp.zeros_like(acc_ref)

    acc_ref[...] += pa + pb

    @pl.when(step == nblocks - 1)
    def _():
        mean = acc_ref[...] * inv_t
        aux_ref[...] = jnp.sum(mean * mean, keepdims=True) * float(N_EXP)


def kernel(x, W):
    b, tt, d = x.shape
    total = b * tt
    half = total // 2
    nblocks = half // BT
    x3 = x.reshape(2, half, d)
    wt = W.T

    body = functools.partial(_router_block, nblocks=nblocks, inv_t=1.0 / total)
    weights, indices, _, aux = pl.pallas_call(
        body,
        grid=(nblocks,),
        in_specs=[
            pl.BlockSpec((1, BT, d), lambda i: (0, i, 0)),
            pl.BlockSpec((1, BT, d), lambda i: (1, i, 0)),
            pl.BlockSpec((d, N_EXP), lambda i: (0, 0)),
        ],
        out_specs=[
            pl.BlockSpec((2, BT, 2), lambda i: (0, i, 0)),
            pl.BlockSpec((2, BT, 2), lambda i: (0, i, 0)),
            pl.BlockSpec((1, N_EXP), lambda i: (0, 0)),
            pl.BlockSpec((1, 1), lambda i: (0, 0)),
        ],
        out_shape=[
            jax.ShapeDtypeStruct((2, half, 2), jnp.float32),
            jax.ShapeDtypeStruct((2, half, 2), jnp.int32),
            jax.ShapeDtypeStruct((1, N_EXP), jnp.float32),
            jax.ShapeDtypeStruct((1, 1), jnp.float32),
        ],
    )(x3, x3, wt)

    return (weights.reshape(b, tt, 2), indices.reshape(b, tt, 2), aux[0, 0])


# speedup vs baseline: 1.4294x; 1.0152x over previous
import functools

import jax
import jax.numpy as jnp
from jax.experimental import pallas as pl
from jax.experimental.pallas import tpu as pltpu

D_MODEL = 2048
N_EXP = 16
BT = 512
NBUF = 4


def _router_block(x_hbm, wt_ref, w_ref, i_ref, acc_ref, aux_ref,
                  buf_ref, sem, *, nsteps, inv_t):
    step = pl.program_id(0)

    def start_copy(src_step, slot):
        pltpu.make_async_copy(
            x_hbm.at[pl.ds(src_step * BT, BT), :],
            buf_ref.at[slot],
            sem.at[slot],
        ).start()

    @pl.when(step == 0)
    def _():
        for j in range(NBUF):
            start_copy(j, j)

    slot = jax.lax.rem(step, NBUF)
    pltpu.make_async_copy(
        x_hbm.at[pl.ds(step * BT, BT), :],
        buf_ref.at[slot],
        sem.at[slot],
    ).wait()

    logits = jnp.dot(buf_ref[slot], wt_ref[...],
                     preferred_element_type=jnp.float32)

    @pl.when(step + NBUF < nsteps)
    def _():
        start_copy(step + NBUF, slot)

    ids = jax.lax.broadcasted_iota(jnp.int32, logits.shape, 1)
    m1 = jnp.max(logits, axis=1, keepdims=True)
    i1 = jnp.min(jnp.where(logits == m1, ids, N_EXP), axis=1, keepdims=True)
    masked = jnp.where(ids == i1, -jnp.inf, logits)
    m2 = jnp.max(masked, axis=1, keepdims=True)
    i2 = jnp.min(jnp.where(masked == m2, ids, N_EXP), axis=1, keepdims=True)

    t = jnp.exp(m2 - m1)
    w1 = 1.0 / (1.0 + t)
    w2 = t / (1.0 + t)
    w_ref[...] = jnp.concatenate([w1, w2], axis=1)
    i_ref[...] = jnp.concatenate([i1, i2], axis=1).astype(jnp.int32)

    p = jnp.exp(logits - m1)
    probs = p / jnp.sum(p, axis=1, keepdims=True)
    part = jnp.sum(probs, axis=0, keepdims=True)

    @pl.when(step == 0)
    def _():
        acc_ref[...] = jnp.zeros_like(acc_ref)

    acc_ref[...] += part

    @pl.when(step == nsteps - 1)
    def _():
        mean = acc_ref[...] * inv_t
        aux_ref[...] = jnp.sum(mean * mean, keepdims=True) * float(N_EXP)


def kernel(x, W):
    b, tt, d = x.shape
    total = b * tt
    xf = x.reshape(total, d)
    wt = W.T
    nsteps = total // BT

    body = functools.partial(_router_block, nsteps=nsteps, inv_t=1.0 / total)
    weights, indices, _, aux = pl.pallas_call(
        body,
        grid=(nsteps,),
        in_specs=[
            pl.BlockSpec(memory_space=pl.ANY),
            pl.BlockSpec((d, N_EXP), lambda i: (0, 0)),
        ],
        out_specs=[
            pl.BlockSpec((BT, 2), lambda i: (i, 0)),
            pl.BlockSpec((BT, 2), lambda i: (i, 0)),
            pl.BlockSpec((1, N_EXP), lambda i: (0, 0)),
            pl.BlockSpec((1, 1), lambda i: (0, 0)),
        ],
        out_shape=[
            jax.ShapeDtypeStruct((total, 2), jnp.float32),
            jax.ShapeDtypeStruct((total, 2), jnp.int32),
            jax.ShapeDtypeStruct((1, N_EXP), jnp.float32),
            jax.ShapeDtypeStruct((1, 1), jnp.float32),
        ],
        scratch_shapes=[
            pltpu.VMEM((NBUF, BT, D_MODEL), jnp.float32),
            pltpu.SemaphoreType.DMA((NBUF,)),
        ],
    )(xf, wt)

    return (weights.reshape(b, tt, 2), indices.reshape(b, tt, 2), aux[0, 0])
